# baseline (speedup 1.0000x reference)
import functools

import jax
import jax.numpy as jnp
import numpy as np
from jax import lax
from jax.experimental import pallas as pl
from jax.experimental.pallas import tpu as pltpu
from jax.experimental.pallas import tpu_sc as plsc

VOCAB = 100000
EMB = 64
CLASSES = 128
BATCH = 1024
TOKENS = 1000
NCORES = 2
NSUB = 16
HROWS = VOCAB
SLICE = HROWS // NSUB
BPT = BATCH // (NCORES * NSUB)
NCHUNK = 8
CH = 125
REAL = 125
RING = 4

_PERM = np.concatenate([
    np.arange(0, 64, 4), np.arange(2, 64, 4),
    np.arange(1, 64, 4), np.arange(3, 64, 4),
])


def _sc_bag_sums(tbl, idx4):
    mesh = plsc.VectorSubcoreMesh(core_axis_name="c", subcore_axis_name="s")

    GB = 4
    NGRP = BPT // GB

    @functools.partial(
        pl.kernel,
        mesh=mesh,
        compiler_params=pltpu.CompilerParams(
            use_tc_tiling_on_sc=False, needs_layout_passes=False
        ),
        out_type=jax.ShapeDtypeStruct((BATCH, EMB), jnp.float32),
        scratch_types=[
            pltpu.VMEM_SHARED((HROWS, EMB), jnp.float8_e4m3fn),
            pltpu.VMEM((2, GB, NCHUNK, CH), jnp.int32),
            pltpu.VMEM((RING, CH, EMB), jnp.float8_e4m3fn),
            pltpu.VMEM((BPT, EMB), jnp.float32),
            [pltpu.SemaphoreType.DMA] * RING,
            [pltpu.SemaphoreType.DMA] * 2,
        ],
    )
    def k(tbl_hbm, idx_hbm, out_hbm, tbl_s, idx_v, rows_v, out_v, sems, isems):
        cid = lax.axis_index("c")
        sid = lax.axis_index("s")

        pltpu.sync_copy(
            tbl_hbm.at[cid, pl.ds(sid * SLICE, SLICE)],
            tbl_s.at[pl.ds(sid * SLICE, SLICE)],
        )

        def idx_issue(g, slot):
            pltpu.async_copy(
                idx_hbm.at[cid, sid, pl.ds(g * GB, GB)],
                idx_v.at[slot],
                isems[slot],
            )

        def idx_wait(g, slot):
            pltpu.make_async_copy(
                idx_hbm.at[cid, sid, pl.ds(g * GB, GB)],
                idx_v.at[slot],
                isems[slot],
            ).wait()

        def issue(slot, b, c, buf):
            pltpu.async_copy(
                tbl_s.at[idx_v.at[slot, b, c]], rows_v.at[buf], sems[buf]
            )

        def wait(slot, b, c, buf):
            pltpu.make_async_copy(
                tbl_s.at[idx_v.at[slot, b, c]], rows_v.at[buf], sems[buf]
            ).wait()

        cmask = jnp.uint32(0xFFFF0000)

        def split_acc(ps, accs, base):
            u = plsc.bitcast(ps, jnp.uint32)
            lo = plsc.bitcast(u << 16, jnp.float32)
            hi = plsc.bitcast(u & cmask, jnp.float32)
            accs[base] = accs[base] + lo
            accs[base + 1] = accs[base + 1] + hi

        def unpack_row(buf, r):
            row = rows_v[buf, r, pl.ds(0, 64)]
            return plsc.unpack(
                row,
                format=plsc.PackFormat.INTERLEAVED,
                preferred_element_type=jnp.bfloat16,
            )

        def accum_chunk(buf, accs):
            def pairs(j, accs):
                r = j * 2
                accs = list(accs)
                a0, b0 = unpack_row(buf, r)
                a1, b1 = unpack_row(buf, r + 1)
                split_acc(a0 + a1, accs, 0)
                split_acc(b0 + b1, accs, 2)
                return tuple(accs)

            accs = lax.fori_loop(0, (REAL - 1) // 2, pairs, accs)
            accs = list(accs)
            a, b = unpack_row(buf, REAL - 1)
            split_acc(a, accs, 0)
            split_acc(b, accs, 2)
            return tuple(accs)

        def bag_body(e, slot, b, nxt, guard=None):
            accs = tuple(jnp.zeros((16,), jnp.float32) for _ in range(4))
            for c in range(NCHUNK):
                buf = c % RING
                wait(slot, b, c, buf)
                accs = accum_chunk(buf, accs)
                if c < NCHUNK - RING:
                    issue(slot, b, c + RING, buf)
                elif nxt is not None:
                    nc = c + RING - NCHUNK
                    if guard is None:
                        issue(nxt[0], nxt[1], nc, buf)
                    else:
                        @pl.when(guard)
                        def _():
                            issue(nxt[0], nxt[1], nc, buf)
            for i in range(4):
                out_v[e, pl.ds(i * 16, 16)] = accs[i]

        idx_issue(0, 0)
        idx_wait(0, 0)
        idx_issue(1, 1)
        plsc.subcore_barrier()
        for c in range(RING):
            issue(0, 0, c, c)

        def main(i, _):
            e0 = i * 2 * GB
            more = i < NGRP // 2 - 1
            for b in range(GB - 1):
                bag_body(e0 + b, 0, b, (0, b + 1))
            idx_wait(2 * i + 1, 1)
            bag_body(e0 + GB - 1, 0, GB - 1, (1, 0))

            @pl.when(more)
            def _():
                idx_issue(2 * i + 2, 0)

            for b in range(GB - 1):
                bag_body(e0 + GB + b, 1, b, (1, b + 1))

            @pl.when(more)
            def _():
                idx_wait(2 * i + 2, 0)
            bag_body(e0 + 2 * GB - 1, 1, GB - 1, (0, 0), guard=more)

            @pl.when(more)
            def _():
                idx_issue(2 * i + 3, 1)
            return 0

        lax.fori_loop(0, NGRP // 2, main, 0)

        pltpu.sync_copy(
            out_v, out_hbm.at[pl.ds((cid * NSUB + sid) * BPT, BPT)]
        )

    return k(tbl, idx4)


def _tc_linear(sums, Wp, b2d):
    def body(x_ref, w_ref, b_ref, o_ref):
        acc = lax.dot_general(
            x_ref[...], w_ref[...],
            (((1,), (1,)), ((), ())),
            preferred_element_type=jnp.float32,
        )
        o_ref[...] = acc * (1.0 / TOKENS) + b_ref[...]

    return pl.pallas_call(
        body,
        out_shape=jax.ShapeDtypeStruct((BATCH, CLASSES), jnp.float32),
    )(sums, Wp, b2d)


def kernel(sents_batch, table, W, b):
    idx4 = sents_batch.reshape(NCORES, NSUB, BPT, NCHUNK, CH).astype(jnp.int32)
    t8 = table.astype(jnp.float8_e4m3fn)
    sums = _sc_bag_sums(jnp.stack([t8, t8]), idx4)
    Wp = W[:, _PERM]
    return _tc_linear(sums, Wp, b.reshape(1, CLASSES))
